# Initial kernel scaffold; baseline (speedup 1.0000x reference)
import jax
import jax.numpy as jnp
from jax.experimental import pallas as pl


def kernel(x, re):
    raise NotImplementedError("write your pallas kernel here")


# speedup vs baseline: 10.3867x; 10.3867x over previous
import functools

import jax
import jax.numpy as jnp
from jax import lax
from jax.experimental import pallas as pl
from jax.experimental.pallas import tpu as pltpu
from jax.experimental.pallas import tpu_sc as plsc

D_MODEL = 128
N_TOKENS = 819200
NUM_CORES = 2
NUM_SUBCORES = 16
NUM_WORKERS = NUM_CORES * NUM_SUBCORES
PER_WORKER = N_TOKENS // NUM_WORKERS
CHUNK = 128
N_CHUNKS = PER_WORKER // CHUNK
NBUF = 2


def _sc_gather_body(table_hbm, idx_hbm, out_hbm, idx_v, rows_v, g0, g1):
    wid = lax.axis_index("s") * NUM_CORES + lax.axis_index("c")
    base = wid * PER_WORKER

    pltpu.sync_copy(idx_hbm.at[wid], idx_v)

    gsems = (g0, g1)

    def start_gather(j, b):
        pltpu.async_copy(table_hbm.at[idx_v.at[j]], rows_v.at[b], gsems[b])

    def wait_gather(j, b):
        pltpu.make_async_copy(table_hbm.at[idx_v.at[j]], rows_v.at[b],
                              gsems[b]).wait()

    start_gather(0, 0)
    start_gather(1, 1)

    @pl.loop(0, N_CHUNKS)
    def _body(j):
        b = lax.rem(j, NBUF)

        def _with_buf(bb):
            wait_gather(j, bb)
            pltpu.sync_copy(rows_v.at[bb],
                            out_hbm.at[pl.ds(base + j * CHUNK, CHUNK)])

            @pl.when(j + NBUF < N_CHUNKS)
            def _():
                start_gather(j + NBUF, bb)

        @pl.when(b == 0)
        def _():
            _with_buf(0)

        @pl.when(b == 1)
        def _():
            _with_buf(1)


@jax.jit
def _sc_gather(x, re):
    idx = x.astype(jnp.int32).reshape(NUM_WORKERS, N_CHUNKS, CHUNK)
    run = pl.kernel(
        _sc_gather_body,
        out_type=jax.ShapeDtypeStruct((N_TOKENS, D_MODEL), jnp.float32),
        mesh=plsc.VectorSubcoreMesh(core_axis_name="c", subcore_axis_name="s"),
        scratch_types=[
            pltpu.VMEM((N_CHUNKS, CHUNK), jnp.int32),
            pltpu.VMEM((NBUF, CHUNK, D_MODEL), jnp.float32),
            pltpu.SemaphoreType.DMA,
            pltpu.SemaphoreType.DMA,
        ],
    )
    return run(re, idx)


def kernel(x, re):
    return _sc_gather(x, re)


# speedup vs baseline: 10.5190x; 1.0127x over previous
import functools

import jax
import jax.numpy as jnp
from jax import lax
from jax.experimental import pallas as pl
from jax.experimental.pallas import tpu as pltpu
from jax.experimental.pallas import tpu_sc as plsc

D_MODEL = 128
N_TOKENS = 819200
NUM_CORES = 2
NUM_SUBCORES = 16
NUM_WORKERS = NUM_CORES * NUM_SUBCORES
PER_WORKER = N_TOKENS // NUM_WORKERS
CHUNK = 128
N_CHUNKS = PER_WORKER // CHUNK
NBUF = 4
LA = 2


def _sc_gather_body(table_hbm, idx_hbm, out_hbm, idx_v, rows_v, *sems):
    gsems, osems = sems[:NBUF], sems[NBUF:]
    wid = lax.axis_index("s") * NUM_CORES + lax.axis_index("c")
    base = wid * PER_WORKER

    pltpu.sync_copy(idx_hbm.at[wid], idx_v)

    def start_gather(j, b):
        pltpu.async_copy(table_hbm.at[idx_v.at[j]], rows_v.at[b], gsems[b])

    def wait_gather(j, b):
        pltpu.make_async_copy(table_hbm.at[idx_v.at[j]], rows_v.at[b],
                              gsems[b]).wait()

    def start_out(j, b):
        pltpu.async_copy(rows_v.at[b],
                         out_hbm.at[pl.ds(base + j * CHUNK, CHUNK)], osems[b])

    def wait_out(j, b):
        pltpu.make_async_copy(rows_v.at[b],
                              out_hbm.at[pl.ds(base + j * CHUNK, CHUNK)],
                              osems[b]).wait()

    for j in range(LA):
        start_gather(j, j)

    @pl.loop(0, N_CHUNKS, step=NBUF)
    def _body(jb):
        for b in range(NBUF):
            j = jb + b
            wait_gather(j, b)
            start_out(j, b)
            jn = j + LA
            bn = (b + LA) % NBUF

            @pl.when(jn < N_CHUNKS)
            def _():
                @pl.when(jn >= NBUF)
                def _():
                    wait_out(jn - NBUF, bn)

                start_gather(jn, bn)

    for b in range(NBUF):
        wait_out(N_CHUNKS - NBUF + b, (N_CHUNKS - NBUF + b) % NBUF)


@jax.jit
def _sc_gather(x, re):
    idx = x.astype(jnp.int32).reshape(NUM_WORKERS, N_CHUNKS, CHUNK)
    run = pl.kernel(
        _sc_gather_body,
        out_type=jax.ShapeDtypeStruct((N_TOKENS, D_MODEL), jnp.float32),
        mesh=plsc.VectorSubcoreMesh(core_axis_name="c", subcore_axis_name="s"),
        scratch_types=(
            [pltpu.VMEM((N_CHUNKS, CHUNK), jnp.int32),
             pltpu.VMEM((NBUF, CHUNK, D_MODEL), jnp.float32)]
            + [pltpu.SemaphoreType.DMA] * (2 * NBUF)
        ),
    )
    return run(re, idx)


def kernel(x, re):
    return _sc_gather(x, re)


# speedup vs baseline: 17.2426x; 1.6392x over previous
import functools

import jax
import jax.numpy as jnp
from jax import lax
from jax.experimental import pallas as pl
from jax.experimental.pallas import tpu as pltpu
from jax.experimental.pallas import tpu_sc as plsc

D_MODEL = 128
N_TOKENS = 819200
NUM_CORES = 2
NUM_SUBCORES = 16
NUM_WORKERS = NUM_CORES * NUM_SUBCORES
PER_WORKER = N_TOKENS // NUM_WORKERS
CHUNK = 128
N_CHUNKS = PER_WORKER // CHUNK
N_PASSES = 2
P_CHUNKS = N_CHUNKS // N_PASSES
NBUF = 2
MAX_LEN_PAD = 9024
STAGE_SUBCORES = 8
STAGE_ROWS = MAX_LEN_PAD // STAGE_SUBCORES


def _sc_gather_body(table_hbm, idx_hbm, out_hbm, tbl_s, idx_v, rows_v,
                    g0, g1, o0, o1):
    gsems = (g0, g1)
    osems = (o0, o1)
    sid = lax.axis_index("s")
    wid = sid * NUM_CORES + lax.axis_index("c")
    base = wid * PER_WORKER

    @pl.when(sid < STAGE_SUBCORES)
    def _():
        pltpu.sync_copy(table_hbm.at[pl.ds(sid * STAGE_ROWS, STAGE_ROWS)],
                        tbl_s.at[pl.ds(sid * STAGE_ROWS, STAGE_ROWS)])

    plsc.subcore_barrier()

    def start_gather(j, b):
        pltpu.async_copy(tbl_s.at[idx_v.at[j]], rows_v.at[b], gsems[b])

    def wait_gather(j, b):
        pltpu.make_async_copy(tbl_s.at[idx_v.at[j]], rows_v.at[b],
                              gsems[b]).wait()

    def start_out(g, b):
        pltpu.async_copy(rows_v.at[b],
                         out_hbm.at[pl.ds(base + g * CHUNK, CHUNK)], osems[b])

    def wait_out(g, b):
        pltpu.make_async_copy(rows_v.at[b],
                              out_hbm.at[pl.ds(base + g * CHUNK, CHUNK)],
                              osems[b]).wait()

    for p in range(N_PASSES):
        pltpu.sync_copy(idx_hbm.at[wid].at[p], idx_v)
        goff = p * P_CHUNKS
        start_gather(0, 0)

        @pl.loop(0, P_CHUNKS, step=NBUF)
        def _body(jb):
            for b in range(NBUF):
                j = jb + b
                wait_gather(j, b)
                start_out(goff + j, b)
                jn = j + 1
                bn = 1 - b

                @pl.when(jn < P_CHUNKS)
                def _():
                    @pl.when(jn >= NBUF)
                    def _():
                        wait_out(goff + jn - NBUF, bn)

                    start_gather(jn, bn)

        wait_out(goff + P_CHUNKS - 2, (P_CHUNKS - 2) % 2)
        wait_out(goff + P_CHUNKS - 1, (P_CHUNKS - 1) % 2)


@jax.jit
def _sc_gather(x, re):
    idx = x.astype(jnp.int32).reshape(NUM_WORKERS, N_PASSES, P_CHUNKS, CHUNK)
    re_pad = jnp.pad(re, ((0, MAX_LEN_PAD - re.shape[0]), (0, 0)))
    run = pl.kernel(
        _sc_gather_body,
        out_type=jax.ShapeDtypeStruct((N_TOKENS, D_MODEL), jnp.float32),
        mesh=plsc.VectorSubcoreMesh(core_axis_name="c", subcore_axis_name="s"),
        scratch_types=(
            [pltpu.VMEM_SHARED((MAX_LEN_PAD, D_MODEL), jnp.float32),
             pltpu.VMEM((P_CHUNKS, CHUNK), jnp.int32),
             pltpu.VMEM((NBUF, CHUNK, D_MODEL), jnp.float32)]
            + [pltpu.SemaphoreType.DMA] * (2 * NBUF)
        ),
    )
    return run(re_pad, idx)


def kernel(x, re):
    return _sc_gather(x, re)


# speedup vs baseline: 17.7996x; 1.0323x over previous
import functools

import jax
import jax.numpy as jnp
from jax import lax
from jax.experimental import pallas as pl
from jax.experimental.pallas import tpu as pltpu
from jax.experimental.pallas import tpu_sc as plsc

D_MODEL = 128
N_TOKENS = 819200
NUM_CORES = 2
NUM_SUBCORES = 16
NUM_WORKERS = NUM_CORES * NUM_SUBCORES
PER_WORKER = N_TOKENS // NUM_WORKERS
CHUNK = 80
N_CHUNKS = PER_WORKER // CHUNK
N_PASSES = 4
P_CHUNKS = N_CHUNKS // N_PASSES
NBUF = 4
LA = 2
MAX_LEN_PAD = 9088
STAGE_ROWS = MAX_LEN_PAD // NUM_SUBCORES


def _sc_gather_body(table_hbm, idx_hbm, out_hbm, tbl_s, idx_v, rows_v, *sems):
    gsems, osems = sems[:NBUF], sems[NBUF:]
    sid = lax.axis_index("s")
    wid = sid * NUM_CORES + lax.axis_index("c")
    base = wid * PER_WORKER

    pltpu.sync_copy(table_hbm.at[pl.ds(sid * STAGE_ROWS, STAGE_ROWS)],
                    tbl_s.at[pl.ds(sid * STAGE_ROWS, STAGE_ROWS)])
    plsc.subcore_barrier()

    def start_gather(j, b):
        pltpu.async_copy(tbl_s.at[idx_v.at[j]], rows_v.at[b], gsems[b])

    def wait_gather(j, b):
        pltpu.make_async_copy(tbl_s.at[idx_v.at[j]], rows_v.at[b],
                              gsems[b]).wait()

    def start_out(g, b):
        pltpu.async_copy(rows_v.at[b],
                         out_hbm.at[pl.ds(base + g * CHUNK, CHUNK)], osems[b])

    def wait_out(g, b):
        pltpu.make_async_copy(rows_v.at[b],
                              out_hbm.at[pl.ds(base + g * CHUNK, CHUNK)],
                              osems[b]).wait()

    for p in range(N_PASSES):
        pltpu.sync_copy(idx_hbm.at[wid].at[p], idx_v)
        goff = p * P_CHUNKS
        for j in range(LA):
            start_gather(j, j)

        @pl.loop(0, P_CHUNKS, step=NBUF)
        def _body(jb):
            for b in range(NBUF):
                j = jb + b
                wait_gather(j, b)
                start_out(goff + j, b)
                jn = j + LA
                bn = (b + LA) % NBUF

                @pl.when(jn < P_CHUNKS)
                def _():
                    @pl.when(jn >= NBUF)
                    def _():
                        wait_out(goff + jn - NBUF, bn)

                    start_gather(jn, bn)

        for b in range(NBUF):
            g = P_CHUNKS - NBUF + b
            wait_out(goff + g, g % NBUF)


@jax.jit
def _sc_gather(x, re):
    idx = x.astype(jnp.int32).reshape(NUM_WORKERS, N_PASSES, P_CHUNKS, CHUNK)
    re_pad = jnp.pad(re, ((0, MAX_LEN_PAD - re.shape[0]), (0, 0)))
    run = pl.kernel(
        _sc_gather_body,
        out_type=jax.ShapeDtypeStruct((N_TOKENS, D_MODEL), jnp.float32),
        mesh=plsc.VectorSubcoreMesh(core_axis_name="c", subcore_axis_name="s"),
        scratch_types=(
            [pltpu.VMEM_SHARED((MAX_LEN_PAD, D_MODEL), jnp.float32),
             pltpu.VMEM((P_CHUNKS, CHUNK), jnp.int32),
             pltpu.VMEM((NBUF, CHUNK, D_MODEL), jnp.float32)]
            + [pltpu.SemaphoreType.DMA] * (2 * NBUF)
        ),
    )
    return run(re_pad, idx)


def kernel(x, re):
    return _sc_gather(x, re)
